# baseline (speedup 1.0000x reference)
import functools

import jax
import jax.numpy as jnp
from jax.experimental import pallas as pl
from jax.experimental.pallas import tpu as pltpu

P = 1048576
CIN = 32
COUT = 32
S = 4

C = 32768
SUB = 16384
NBUF = 6
NOBUF = 6


def _body(ld_ref, lev_hbm, x_hbm, w_ref, b_ref, o_hbm,
          xbuf, lbuf, obuf, x_sems, l_sems, o_sems):
    num = x_hbm.shape[2] // C
    i = pl.program_id(0)

    def start_in(c, slot):
        pltpu.make_async_copy(
            x_hbm.at[0, :, pl.ds(c * C, C)], xbuf.at[slot], x_sems.at[slot]
        ).start()

    @pl.when(i == 0)
    def _():
        pltpu.make_async_copy(lev_hbm, lbuf, l_sems).start()
        for k in range(min(NBUF, num)):
            start_in(k, k)
        pltpu.make_async_copy(lev_hbm, lbuf, l_sems).wait()

    slot = jax.lax.rem(i, NBUF)
    oslot = jax.lax.rem(i, NOBUF)

    pltpu.make_async_copy(
        x_hbm.at[0, :, pl.ds(0, C)], xbuf.at[slot], x_sems.at[slot]).wait()

    @pl.when(i >= NOBUF)
    def _():
        pltpu.make_async_copy(
            obuf.at[oslot], o_hbm.at[0, :, pl.ds(0, C)], o_sems.at[oslot]
        ).wait()

    delta = ld_ref[0]
    for c in range(0, C, SUB):
        sl = pl.ds(c, SUB)
        s = jnp.clip(lbuf[:, pl.ds(i * C + c, SUB)] + delta, 0, S - 1)
        y = jnp.dot(w_ref[:], xbuf[slot, :, sl],
                    preferred_element_type=jnp.float32)
        ya = jnp.where(s == 0, y[0:COUT, :], y[COUT:2 * COUT, :])
        yb = jnp.where(s == 2, y[2 * COUT:3 * COUT, :], y[3 * COUT:, :])
        obuf[oslot, :, sl] = jnp.where(s <= 1, ya, yb) + b_ref[:]

    pltpu.make_async_copy(
        obuf.at[oslot], o_hbm.at[0, :, pl.ds(i * C, C)], o_sems.at[oslot]
    ).start()

    @pl.when(i + NBUF < num)
    def _():
        start_in(i + NBUF, slot)

    @pl.when(i == num - 1)
    def _():
        for k in range(NOBUF):
            pltpu.make_async_copy(
                obuf.at[k], o_hbm.at[0, :, pl.ds(0, C)], o_sems.at[k]
            ).wait()


@jax.jit
def _run(x, levels2d, level_deltas, wstack, bias2d):
    p = x.shape[2]
    num = p // C
    return pl.pallas_call(
        _body,
        grid=(num,),
        in_specs=[
            pl.BlockSpec(memory_space=pltpu.SMEM),
            pl.BlockSpec(memory_space=pltpu.MemorySpace.HBM),
            pl.BlockSpec(memory_space=pltpu.MemorySpace.HBM),
            pl.BlockSpec((S * COUT, CIN), lambda i: (0, 0)),
            pl.BlockSpec((COUT, 1), lambda i: (0, 0)),
        ],
        out_specs=pl.BlockSpec(memory_space=pltpu.MemorySpace.HBM),
        out_shape=jax.ShapeDtypeStruct((1, COUT, p), x.dtype),
        scratch_shapes=[
            pltpu.VMEM((NBUF, CIN, C), x.dtype),
            pltpu.VMEM((1, p), jnp.int32),
            pltpu.VMEM((NOBUF, COUT, C), x.dtype),
            pltpu.SemaphoreType.DMA((NBUF,)),
            pltpu.SemaphoreType.DMA,
            pltpu.SemaphoreType.DMA((NOBUF,)),
        ],
        compiler_params=pltpu.CompilerParams(
            dimension_semantics=("arbitrary",),
        ),
    )(level_deltas, levels2d, x, wstack, bias2d)


def kernel(input_features, levels, level_deltas, weight, bias):
    wstack = weight.reshape(S * COUT, CIN)
    levels2d = levels.reshape(1, -1)
    bias2d = bias.reshape(COUT, 1)
    return _run(input_features, levels2d, level_deltas, wstack, bias2d)


# speedup vs baseline: 1.0352x; 1.0352x over previous
import jax
import jax.numpy as jnp
from jax.experimental import pallas as pl
from jax.experimental.pallas import tpu as pltpu

P = 1048576
CIN = 32
COUT = 32
S = 4

C = 32768
SUB = 16384
NBUF = 6
NOBUF = 6


def _body(ld_ref, lev_hbm, x_hbm, w_ref, b_ref, o_hbm,
          xbuf, lbuf, obuf, x_sems, l_sems, o_sems):
    num = x_hbm.shape[2] // C
    i = pl.program_id(0)

    def start_in(c, slot):
        pltpu.make_async_copy(
            x_hbm.at[0, :, pl.ds(c * C, C)], xbuf.at[slot], x_sems.at[slot]
        ).start()
        pltpu.make_async_copy(
            lev_hbm.at[:, pl.ds(c * C, C)], lbuf.at[slot], l_sems.at[slot]
        ).start()

    @pl.when(i == 0)
    def _():
        for k in range(min(NBUF, num)):
            start_in(k, k)

    slot = jax.lax.rem(i, NBUF)
    oslot = jax.lax.rem(i, NOBUF)

    pltpu.make_async_copy(
        x_hbm.at[0, :, pl.ds(0, C)], xbuf.at[slot], x_sems.at[slot]).wait()
    pltpu.make_async_copy(
        lev_hbm.at[:, pl.ds(0, C)], lbuf.at[slot], l_sems.at[slot]).wait()

    @pl.when(i >= NOBUF)
    def _():
        pltpu.make_async_copy(
            obuf.at[oslot], o_hbm.at[0, :, pl.ds(0, C)], o_sems.at[oslot]
        ).wait()

    delta = ld_ref[0]
    for c in range(0, C, SUB):
        sl = pl.ds(c, SUB)
        s = jnp.clip(lbuf[slot, :, sl] + delta, 0, S - 1)
        y = jnp.dot(w_ref[:], xbuf[slot, :, sl],
                    preferred_element_type=jnp.float32)
        ya = jnp.where(s == 0, y[0:COUT, :], y[COUT:2 * COUT, :])
        yb = jnp.where(s == 2, y[2 * COUT:3 * COUT, :], y[3 * COUT:, :])
        obuf[oslot, :, sl] = jnp.where(s <= 1, ya, yb) + b_ref[:]

    pltpu.make_async_copy(
        obuf.at[oslot], o_hbm.at[0, :, pl.ds(i * C, C)], o_sems.at[oslot]
    ).start()

    @pl.when(i + NBUF < num)
    def _():
        start_in(i + NBUF, slot)

    @pl.when(i == num - 1)
    def _():
        for k in range(NOBUF):
            pltpu.make_async_copy(
                obuf.at[k], o_hbm.at[0, :, pl.ds(0, C)], o_sems.at[k]
            ).wait()


@jax.jit
def _run(x, levels2d, level_deltas, wstack, bias2d):
    p = x.shape[2]
    num = p // C
    return pl.pallas_call(
        _body,
        grid=(num,),
        in_specs=[
            pl.BlockSpec(memory_space=pltpu.SMEM),
            pl.BlockSpec(memory_space=pltpu.MemorySpace.HBM),
            pl.BlockSpec(memory_space=pltpu.MemorySpace.HBM),
            pl.BlockSpec((S * COUT, CIN), lambda i: (0, 0)),
            pl.BlockSpec((COUT, 1), lambda i: (0, 0)),
        ],
        out_specs=pl.BlockSpec(memory_space=pltpu.MemorySpace.HBM),
        out_shape=jax.ShapeDtypeStruct((1, COUT, p), x.dtype),
        scratch_shapes=[
            pltpu.VMEM((NBUF, CIN, C), x.dtype),
            pltpu.VMEM((NBUF, 1, C), jnp.int32),
            pltpu.VMEM((NOBUF, COUT, C), x.dtype),
            pltpu.SemaphoreType.DMA((NBUF,)),
            pltpu.SemaphoreType.DMA((NBUF,)),
            pltpu.SemaphoreType.DMA((NOBUF,)),
        ],
        compiler_params=pltpu.CompilerParams(
            dimension_semantics=("arbitrary",),
        ),
    )(level_deltas, levels2d, x, wstack, bias2d)


def kernel(input_features, levels, level_deltas, weight, bias):
    wstack = weight.reshape(S * COUT, CIN)
    levels2d = levels.reshape(1, -1)
    bias2d = bias.reshape(COUT, 1)
    return _run(input_features, levels2d, level_deltas, wstack, bias2d)
